# baseline (speedup 1.0000x reference)
import functools

import jax
import jax.numpy as jnp
from jax import lax
from jax.experimental import pallas as pl
from jax.experimental.pallas import tpu as pltpu
from jax.experimental.pallas import tpu_sc as plsc

_NUM_FIELDS = 26
_FIELD_DIM = 40000
_BUCKET = 100000
_D = 64
_QPR = 11
_B = 4096
_N = _B * _NUM_FIELDS

_NC = 2
_NS = 16
_NW = _NC * _NS
_L = 16
_ROWS_PER_W = _N // _NW
_CHUNK = 128
_NCHUNKS = _ROWS_PER_W // _CHUNK

_C0 = 0.00999890307480919
_C1 = 0.0050208361432609455
_C2 = -8.485290183949355e-05
_C3 = -0.0003149865897275671


def _sc_body(x_hbm, qv_hbm, rv_hbm, qs_hbm, rs_hbm, out_hbm,
             xall_v, idxq_v, idxr_v,
             qv_a, qs_a, rv_a, rs_a, out_a,
             qv_b, qs_b, rv_b, rs_b, out_b,
             sem_a, sem_b, sem_oa, sem_ob):
    wid = lax.axis_index("s") * _NC + lax.axis_index("c")
    base = wid * _ROWS_PER_W

    pltpu.sync_copy(x_hbm.at[pl.ds(base, _ROWS_PER_W)], xall_v)
    lane = lax.iota(jnp.int32, _L)

    def idx_body(j, carry):
        xv = xall_v[pl.ds(j * _L, _L)]
        col = lax.rem(base + j * _L + lane, _NUM_FIELDS)
        xn = xv + col * _FIELD_DIM
        idxq_v[pl.ds(j * _L, _L)] = lax.div(xn, _QPR)
        idxr_v[pl.ds(j * _L, _L)] = lax.rem(xn, _BUCKET)
        return carry

    lax.fori_loop(0, _ROWS_PER_W // _L, idx_body, 0, unroll=2)

    def fire_gather(c, qv_v, qs_v, rv_v, rs_v, sem):
        iq = idxq_v.at[pl.ds(c * _CHUNK, _CHUNK)]
        ir = idxr_v.at[pl.ds(c * _CHUNK, _CHUNK)]
        pltpu.async_copy(qv_hbm.at[iq], qv_v, sem)
        pltpu.async_copy(qs_hbm.at[iq], qs_v, sem)
        pltpu.async_copy(rv_hbm.at[ir], rv_v, sem)
        pltpu.async_copy(rs_hbm.at[ir], rs_v, sem)

    def wait_gather(c, qv_v, qs_v, rv_v, rs_v, sem):
        iq = idxq_v.at[pl.ds(c * _CHUNK, _CHUNK)]
        ir = idxr_v.at[pl.ds(c * _CHUNK, _CHUNK)]
        pltpu.make_async_copy(qv_hbm.at[iq], qv_v, sem).wait()
        pltpu.make_async_copy(qs_hbm.at[iq], qs_v, sem).wait()
        pltpu.make_async_copy(rv_hbm.at[ir], rv_v, sem).wait()
        pltpu.make_async_copy(rs_hbm.at[ir], rs_v, sem).wait()

    def compute(qv_v, qs_v, rv_v, rs_v, out_v):
        def row_body(i, carry):
            for k in range(_D // _L):
                sl = pl.ds(k * _L, _L)
                sq = qs_v[i, sl]
                sr = rs_v[i, sl]
                tq = _C0 + sq * (_C1 + sq * (_C2 + sq * _C3))
                tr = _C0 + sr * (_C1 + sr * (_C2 + sr * _C3))
                vq = qv_v[i, sl]
                vr = rv_v[i, sl]
                pq = vq - jnp.minimum(jnp.maximum(vq, -tq), tq)
                pr = vr - jnp.minimum(jnp.maximum(vr, -tr), tr)
                out_v[i, sl] = pq + pr
            return carry

        lax.fori_loop(0, _CHUNK, row_body, 0, unroll=2)

    def store_slice(c, out_v, sem):
        return pltpu.make_async_copy(
            out_v, out_hbm.at[pl.ds(base + c * _CHUNK, _CHUNK)], sem)

    def fire_store(c, out_v, sem):
        pltpu.async_copy(out_v, out_hbm.at[pl.ds(base + c * _CHUNK, _CHUNK)],
                         sem)

    bufs_a = (qv_a, qs_a, rv_a, rs_a)
    bufs_b = (qv_b, qs_b, rv_b, rs_b)

    fire_gather(0, *bufs_a, sem_a)
    fire_gather(1, *bufs_b, sem_b)

    def step(g, carry):
        c_a = 2 * g
        c_b = c_a + 1

        wait_gather(c_a, *bufs_a, sem_a)
        compute(*bufs_a, out_a)

        @pl.when(g > 0)
        def _():
            store_slice(c_a - 2, out_a, sem_oa).wait()

        fire_store(c_a, out_a, sem_oa)

        @pl.when(g < (_NCHUNKS // 2 - 1))
        def _():
            fire_gather(c_a + 2, *bufs_a, sem_a)

        wait_gather(c_b, *bufs_b, sem_b)
        compute(*bufs_b, out_b)

        @pl.when(g > 0)
        def _():
            store_slice(c_b - 2, out_b, sem_ob).wait()

        fire_store(c_b, out_b, sem_ob)

        @pl.when(g < (_NCHUNKS // 2 - 1))
        def _():
            fire_gather(c_b + 2, *bufs_b, sem_b)

        return carry

    lax.fori_loop(0, _NCHUNKS // 2, step, 0)

    store_slice(_NCHUNKS - 2, out_a, sem_oa).wait()
    store_slice(_NCHUNKS - 1, out_b, sem_ob).wait()


_mesh = plsc.VectorSubcoreMesh(core_axis_name="c", subcore_axis_name="s")

_ce_kernel = functools.partial(
    pl.kernel,
    out_type=jax.ShapeDtypeStruct((_N, _D), jnp.float32),
    mesh=_mesh,
    scratch_types=[
        pltpu.VMEM((_ROWS_PER_W,), jnp.int32),
        pltpu.VMEM((_ROWS_PER_W,), jnp.int32),
        pltpu.VMEM((_ROWS_PER_W,), jnp.int32),
        pltpu.VMEM((_CHUNK, _D), jnp.float32),
        pltpu.VMEM((_CHUNK, _D), jnp.float32),
        pltpu.VMEM((_CHUNK, _D), jnp.float32),
        pltpu.VMEM((_CHUNK, _D), jnp.float32),
        pltpu.VMEM((_CHUNK, _D), jnp.float32),
        pltpu.VMEM((_CHUNK, _D), jnp.float32),
        pltpu.VMEM((_CHUNK, _D), jnp.float32),
        pltpu.VMEM((_CHUNK, _D), jnp.float32),
        pltpu.VMEM((_CHUNK, _D), jnp.float32),
        pltpu.VMEM((_CHUNK, _D), jnp.float32),
        pltpu.SemaphoreType.DMA,
        pltpu.SemaphoreType.DMA,
        pltpu.SemaphoreType.DMA,
        pltpu.SemaphoreType.DMA,
    ],
    compiler_params=pltpu.CompilerParams(use_tc_tiling_on_sc=False),
)(_sc_body)


def kernel(x, Q_v, R_v, Q_s, R_s):
    x_flat = x.reshape(_N)
    out = _ce_kernel(x_flat, Q_v, R_v, Q_s, R_s)
    return out.reshape(_B, _NUM_FIELDS, _D)


# speedup vs baseline: 1.0546x; 1.0546x over previous
import functools

import jax
import jax.numpy as jnp
from jax import lax
from jax.experimental import pallas as pl
from jax.experimental.pallas import tpu as pltpu
from jax.experimental.pallas import tpu_sc as plsc

_NUM_FIELDS = 26
_FIELD_DIM = 40000
_BUCKET = 100000
_D = 64
_QPR = 11
_B = 4096
_N = _B * _NUM_FIELDS

_NC = 2
_NS = 16
_NW = _NC * _NS
_L = 16
_ROWS_PER_W = _N // _NW
_CHUNK = 128
_NCHUNKS = _ROWS_PER_W // _CHUNK

_C0 = 0.00999890307480919
_C1 = 0.0050208361432609455
_C2 = -8.485290183949355e-05
_C3 = -0.0003149865897275671


def _sc_body(x_hbm, qc_hbm, rc_hbm, out_hbm,
             xall_v, idxq_v, idxr_v,
             qc_a, rc_a, o_a, qc_b, rc_b, o_b,
             sem_a, sem_b, sem_oa, sem_ob):
    wid = lax.axis_index("s") * _NC + lax.axis_index("c")
    base = wid * _ROWS_PER_W
    obase = base // 2

    pltpu.sync_copy(x_hbm.at[pl.ds(base, _ROWS_PER_W)], xall_v)
    lane = lax.iota(jnp.int32, _L)

    def idx_body(j, carry):
        xv = xall_v[pl.ds(j * _L, _L)]
        col = lax.rem(base + j * _L + lane, _NUM_FIELDS)
        xn = xv + col * _FIELD_DIM
        idxq_v[pl.ds(j * _L, _L)] = lax.div(xn, _QPR)
        idxr_v[pl.ds(j * _L, _L)] = lax.rem(xn, _BUCKET)
        return carry

    lax.fori_loop(0, _ROWS_PER_W // _L, idx_body, 0, unroll=4)

    def fire_gather(c, qc_v, rc_v, sem):
        iq = idxq_v.at[pl.ds(c * _CHUNK, _CHUNK)]
        ir = idxr_v.at[pl.ds(c * _CHUNK, _CHUNK)]
        pltpu.async_copy(qc_hbm.at[iq], qc_v, sem)
        pltpu.async_copy(rc_hbm.at[ir], rc_v, sem)

    def wait_gather(c, qc_v, rc_v, sem):
        iq = idxq_v.at[pl.ds(c * _CHUNK, _CHUNK)]
        ir = idxr_v.at[pl.ds(c * _CHUNK, _CHUNK)]
        pltpu.make_async_copy(qc_hbm.at[iq], qc_v, sem).wait()
        pltpu.make_async_copy(rc_hbm.at[ir], rc_v, sem).wait()

    def _prune_slice(v, s):
        t = _C0 + s * (_C1 + s * (_C2 + s * _C3))
        return v - jnp.minimum(jnp.maximum(v, -t), t)

    def compute(qc_v, rc_v, o_v):
        def pair_body(u, carry):
            i = 2 * u
            for h in range(2):
                for k in range(_D // _L):
                    src = pl.ds(k * _L, _L)
                    ssl = pl.ds(_D + k * _L, _L)
                    pq = _prune_slice(qc_v[i + h, src], qc_v[i + h, ssl])
                    pr = _prune_slice(rc_v[i + h, src], rc_v[i + h, ssl])
                    o_v[u, pl.ds(h * _D + k * _L, _L)] = pq + pr
            return carry

        lax.fori_loop(0, _CHUNK // 2, pair_body, 0, unroll=2)

    def store_slice(c, o_v, sem):
        off = pl.multiple_of(obase + c * (_CHUNK // 2), 8)
        dst = out_hbm.at[pl.ds(off, _CHUNK // 2)]
        return pltpu.make_async_copy(o_v, dst, sem)

    fire_gather(0, qc_a, rc_a, sem_a)
    fire_gather(1, qc_b, rc_b, sem_b)

    def step(g, carry):
        c_a = 2 * g
        c_b = c_a + 1

        wait_gather(c_a, qc_a, rc_a, sem_a)
        compute(qc_a, rc_a, o_a)

        @pl.when(g > 0)
        def _():
            store_slice(c_a - 2, o_a, sem_oa).wait()

        store_slice(c_a, o_a, sem_oa).start()

        @pl.when(g < (_NCHUNKS // 2 - 1))
        def _():
            fire_gather(c_a + 2, qc_a, rc_a, sem_a)

        wait_gather(c_b, qc_b, rc_b, sem_b)
        compute(qc_b, rc_b, o_b)

        @pl.when(g > 0)
        def _():
            store_slice(c_b - 2, o_b, sem_ob).wait()

        store_slice(c_b, o_b, sem_ob).start()

        @pl.when(g < (_NCHUNKS // 2 - 1))
        def _():
            fire_gather(c_b + 2, qc_b, rc_b, sem_b)

        return carry

    lax.fori_loop(0, _NCHUNKS // 2, step, 0)

    store_slice(_NCHUNKS - 2, o_a, sem_oa).wait()
    store_slice(_NCHUNKS - 1, o_b, sem_ob).wait()


_mesh = plsc.VectorSubcoreMesh(core_axis_name="c", subcore_axis_name="s")

_ce_kernel = functools.partial(
    pl.kernel,
    out_type=jax.ShapeDtypeStruct((_N // 2, 2 * _D), jnp.float32),
    mesh=_mesh,
    scratch_types=[
        pltpu.VMEM((_ROWS_PER_W,), jnp.int32),
        pltpu.VMEM((_ROWS_PER_W,), jnp.int32),
        pltpu.VMEM((_ROWS_PER_W,), jnp.int32),
        pltpu.VMEM((_CHUNK, 2 * _D), jnp.float32),
        pltpu.VMEM((_CHUNK, 2 * _D), jnp.float32),
        pltpu.VMEM((_CHUNK // 2, 2 * _D), jnp.float32),
        pltpu.VMEM((_CHUNK, 2 * _D), jnp.float32),
        pltpu.VMEM((_CHUNK, 2 * _D), jnp.float32),
        pltpu.VMEM((_CHUNK // 2, 2 * _D), jnp.float32),
        pltpu.SemaphoreType.DMA,
        pltpu.SemaphoreType.DMA,
        pltpu.SemaphoreType.DMA,
        pltpu.SemaphoreType.DMA,
    ],
)(_sc_body)


def kernel(x, Q_v, R_v, Q_s, R_s):
    qc = jnp.concatenate([Q_v, Q_s], axis=1)
    rc = jnp.concatenate([R_v, R_s], axis=1)
    x_flat = x.reshape(_N)
    out = _ce_kernel(x_flat, qc, rc)
    return out.reshape(_B, _NUM_FIELDS, _D)


# speedup vs baseline: 1.3130x; 1.2450x over previous
import functools

import jax
import jax.numpy as jnp
from jax import lax
from jax.experimental import pallas as pl
from jax.experimental.pallas import tpu as pltpu
from jax.experimental.pallas import tpu_sc as plsc

_NUM_FIELDS = 26
_FIELD_DIM = 40000
_BUCKET = 100000
_D = 64
_GK = 0.02
_QPR = 11
_B = 4096
_N = _B * _NUM_FIELDS


_TC_BLK = 2000


def _prune_body(qv_ref, qs_ref, rv_ref, rs_ref, qd_ref, rd_ref):
    for v_ref, s_ref, o_ref in ((qv_ref, qs_ref, qd_ref),
                                (rv_ref, rs_ref, rd_ref)):
        v = v_ref[...]
        t = _GK * jax.nn.sigmoid(s_ref[...])
        p = v - jnp.minimum(jnp.maximum(v, -t), t)
        o_ref[...] = jnp.concatenate([p, p], axis=1)


_prune_tables = pl.pallas_call(
    _prune_body,
    grid=(_BUCKET // _TC_BLK,),
    in_specs=[pl.BlockSpec((_TC_BLK, _D), lambda i: (i, 0))] * 4,
    out_specs=[pl.BlockSpec((_TC_BLK, 2 * _D), lambda i: (i, 0))] * 2,
    out_shape=[jax.ShapeDtypeStruct((_BUCKET, 2 * _D), jnp.float32)] * 2,
)


_NC = 2
_NS = 16
_NW = _NC * _NS
_L = 16
_BATCH_PER_W = _B // _NW
_ROWS_PER_W = _N // _NW
_NB = 4
_CHUNK = _NB * _NUM_FIELDS
_NCHUNKS = _BATCH_PER_W // _NB


def _sc_body(x_hbm, qd_hbm, rd_hbm, out_hbm,
             xall_v, idxq_v, idxr_v,
             qc_a, rc_a, o_a, qc_b, rc_b, o_b,
             sem_a, sem_b, sem_oa, sem_ob):
    wid = lax.axis_index("s") * _NC + lax.axis_index("c")
    base = wid * _ROWS_PER_W
    bbase = wid * _BATCH_PER_W

    pltpu.sync_copy(x_hbm.at[pl.ds(base, _ROWS_PER_W)], xall_v)
    lane = lax.iota(jnp.int32, _L)

    def idx_body(j, carry):
        xv = xall_v[pl.ds(j * _L, _L)]
        col = lax.rem(base + j * _L + lane, _NUM_FIELDS)
        xn = xv + col * _FIELD_DIM
        idxq_v[pl.ds(j * _L, _L)] = lax.div(xn, _QPR)
        idxr_v[pl.ds(j * _L, _L)] = lax.rem(xn, _BUCKET)
        return carry

    lax.fori_loop(0, _ROWS_PER_W // _L, idx_body, 0, unroll=4)

    def fire_gather(c, qc_v, rc_v, sem):
        iq = idxq_v.at[pl.ds(c * _CHUNK, _CHUNK)]
        ir = idxr_v.at[pl.ds(c * _CHUNK, _CHUNK)]
        pltpu.async_copy(qd_hbm.at[iq], qc_v, sem)
        pltpu.async_copy(rd_hbm.at[ir], rc_v, sem)

    def wait_gather(c, qc_v, rc_v, sem):
        iq = idxq_v.at[pl.ds(c * _CHUNK, _CHUNK)]
        ir = idxr_v.at[pl.ds(c * _CHUNK, _CHUNK)]
        pltpu.make_async_copy(qd_hbm.at[iq], qc_v, sem).wait()
        pltpu.make_async_copy(rd_hbm.at[ir], rc_v, sem).wait()

    def compute(qc_v, rc_v, o_v):
        def b_body(b, carry):
            def f_body(f, carry2):
                i = b * _NUM_FIELDS + f
                for k in range(_D // _L):
                    sl = pl.ds(k * _L, _L)
                    o_v[b, f, sl] = qc_v[i, sl] + rc_v[i, sl]
                return carry2

            lax.fori_loop(0, _NUM_FIELDS, f_body, 0, unroll=2)
            return carry

        lax.fori_loop(0, _NB, b_body, 0)

    def store_slice(c, o_v, sem):
        dst = out_hbm.at[pl.ds(bbase + c * _NB, _NB)]
        return pltpu.make_async_copy(o_v, dst, sem)

    fire_gather(0, qc_a, rc_a, sem_a)
    fire_gather(1, qc_b, rc_b, sem_b)

    def step(g, carry):
        c_a = 2 * g
        c_b = c_a + 1

        wait_gather(c_a, qc_a, rc_a, sem_a)
        compute(qc_a, rc_a, o_a)

        @pl.when(g > 0)
        def _():
            store_slice(c_a - 2, o_a, sem_oa).wait()

        store_slice(c_a, o_a, sem_oa).start()

        @pl.when(g < (_NCHUNKS // 2 - 1))
        def _():
            fire_gather(c_a + 2, qc_a, rc_a, sem_a)

        wait_gather(c_b, qc_b, rc_b, sem_b)
        compute(qc_b, rc_b, o_b)

        @pl.when(g > 0)
        def _():
            store_slice(c_b - 2, o_b, sem_ob).wait()

        store_slice(c_b, o_b, sem_ob).start()

        @pl.when(g < (_NCHUNKS // 2 - 1))
        def _():
            fire_gather(c_b + 2, qc_b, rc_b, sem_b)

        return carry

    lax.fori_loop(0, _NCHUNKS // 2, step, 0)

    store_slice(_NCHUNKS - 2, o_a, sem_oa).wait()
    store_slice(_NCHUNKS - 1, o_b, sem_ob).wait()


_mesh = plsc.VectorSubcoreMesh(core_axis_name="c", subcore_axis_name="s")

_ce_kernel = functools.partial(
    pl.kernel,
    out_type=jax.ShapeDtypeStruct((_B, _NUM_FIELDS, _D), jnp.float32),
    mesh=_mesh,
    scratch_types=[
        pltpu.VMEM((_ROWS_PER_W,), jnp.int32),
        pltpu.VMEM((_ROWS_PER_W,), jnp.int32),
        pltpu.VMEM((_ROWS_PER_W,), jnp.int32),
        pltpu.VMEM((_CHUNK, 2 * _D), jnp.float32),
        pltpu.VMEM((_CHUNK, 2 * _D), jnp.float32),
        pltpu.VMEM((_NB, _NUM_FIELDS, _D), jnp.float32),
        pltpu.VMEM((_CHUNK, 2 * _D), jnp.float32),
        pltpu.VMEM((_CHUNK, 2 * _D), jnp.float32),
        pltpu.VMEM((_NB, _NUM_FIELDS, _D), jnp.float32),
        pltpu.SemaphoreType.DMA,
        pltpu.SemaphoreType.DMA,
        pltpu.SemaphoreType.DMA,
        pltpu.SemaphoreType.DMA,
    ],
)(_sc_body)


def kernel(x, Q_v, R_v, Q_s, R_s):
    qd, rd = _prune_tables(Q_v, Q_s, R_v, R_s)
    x_flat = x.reshape(_N)
    return _ce_kernel(x_flat, qd, rd)


# speedup vs baseline: 1.3153x; 1.0017x over previous
import functools

import jax
import jax.numpy as jnp
from jax import lax
from jax.experimental import pallas as pl
from jax.experimental.pallas import tpu as pltpu
from jax.experimental.pallas import tpu_sc as plsc

_NUM_FIELDS = 26
_FIELD_DIM = 40000
_BUCKET = 100000
_D = 64
_GK = 0.02
_QPR = 11
_B = 4096
_N = _B * _NUM_FIELDS


_TC_BLK = 2000


def _prune_body(qv_ref, qs_ref, rv_ref, rs_ref, qd_ref, rd_ref):
    for v_ref, s_ref, o_ref in ((qv_ref, qs_ref, qd_ref),
                                (rv_ref, rs_ref, rd_ref)):
        v = v_ref[...]
        t = _GK * jax.nn.sigmoid(s_ref[...])
        p = v - jnp.minimum(jnp.maximum(v, -t), t)
        o_ref[...] = jnp.concatenate([p, p], axis=1)


_prune_tables = pl.pallas_call(
    _prune_body,
    grid=(_BUCKET // _TC_BLK,),
    in_specs=[pl.BlockSpec((_TC_BLK, _D), lambda i: (i, 0))] * 4,
    out_specs=[pl.BlockSpec((_TC_BLK, 2 * _D), lambda i: (i, 0))] * 2,
    out_shape=[jax.ShapeDtypeStruct((_BUCKET, 2 * _D), jnp.float32)] * 2,
)


_NC = 2
_NS = 16
_NW = _NC * _NS
_L = 16
_BATCH_PER_W = _B // _NW
_ROWS_PER_W = _N // _NW
_NB = 4
_CHUNK = _NB * _NUM_FIELDS
_NCHUNKS = _BATCH_PER_W // _NB


def _sc_body(x_hbm, qd_hbm, rd_hbm, out_hbm,
             xall_v, idxq_v, idxr_v,
             qc_a, rc_a, o_a, qc_b, rc_b, o_b,
             sem_a, sem_b, sem_oa, sem_ob):
    wid = lax.axis_index("s") * _NC + lax.axis_index("c")
    base = wid * _ROWS_PER_W
    bbase = wid * _BATCH_PER_W

    pltpu.sync_copy(x_hbm.at[pl.ds(base, _ROWS_PER_W)], xall_v)
    lane = lax.iota(jnp.int32, _L)

    def idx_body(j, carry):
        xv = xall_v[pl.ds(j * _L, _L)]
        col = lax.rem(base + j * _L + lane, _NUM_FIELDS)
        xn = xv + col * _FIELD_DIM
        idxq_v[pl.ds(j * _L, _L)] = lax.div(xn, _QPR)
        idxr_v[pl.ds(j * _L, _L)] = lax.rem(xn, _BUCKET)
        return carry

    lax.fori_loop(0, _ROWS_PER_W // _L, idx_body, 0, unroll=4)

    def fire_gather(c, qc_v, rc_v, sem):
        iq = idxq_v.at[pl.ds(c * _CHUNK, _CHUNK)]
        ir = idxr_v.at[pl.ds(c * _CHUNK, _CHUNK)]
        pltpu.async_copy(qd_hbm.at[iq], qc_v, sem)
        pltpu.async_copy(rd_hbm.at[ir], rc_v, sem)

    def wait_gather(c, qc_v, rc_v, sem):
        iq = idxq_v.at[pl.ds(c * _CHUNK, _CHUNK)]
        ir = idxr_v.at[pl.ds(c * _CHUNK, _CHUNK)]
        pltpu.make_async_copy(qd_hbm.at[iq], qc_v, sem).wait()
        pltpu.make_async_copy(rd_hbm.at[ir], rc_v, sem).wait()

    def compute(qc_v, rc_v, o_v):
        def b_body(b, carry):
            def f_body(f, carry2):
                i = b * _NUM_FIELDS + f
                for k in range(_D // _L):
                    sl = pl.ds(k * _L, _L)
                    o_v[b, f, sl] = qc_v[i, sl] + rc_v[i, sl]
                return carry2

            lax.fori_loop(0, _NUM_FIELDS, f_body, 0, unroll=2)
            return carry

        lax.fori_loop(0, _NB, b_body, 0)

    def store_slice(c, o_v, sem):
        dst = out_hbm.at[pl.ds(bbase + c * _NB, _NB)]
        return pltpu.make_async_copy(o_v, dst, sem)

    fire_gather(0, qc_a, rc_a, sem_a)
    fire_gather(1, qc_b, rc_b, sem_b)

    def step(g, carry):
        c_a = 2 * g
        c_b = c_a + 1

        wait_gather(c_a, qc_a, rc_a, sem_a)
        compute(qc_a, rc_a, o_a)

        @pl.when(g > 0)
        def _():
            store_slice(c_a - 2, o_a, sem_oa).wait()

        store_slice(c_a, o_a, sem_oa).start()

        @pl.when(g < (_NCHUNKS // 2 - 1))
        def _():
            fire_gather(c_a + 2, qc_a, rc_a, sem_a)

        wait_gather(c_b, qc_b, rc_b, sem_b)
        compute(qc_b, rc_b, o_b)

        @pl.when(g > 0)
        def _():
            store_slice(c_b - 2, o_b, sem_ob).wait()

        store_slice(c_b, o_b, sem_ob).start()

        @pl.when(g < (_NCHUNKS // 2 - 1))
        def _():
            fire_gather(c_b + 2, qc_b, rc_b, sem_b)

        return carry

    lax.fori_loop(0, _NCHUNKS // 2, step, 0)

    store_slice(_NCHUNKS - 2, o_a, sem_oa).wait()
    store_slice(_NCHUNKS - 1, o_b, sem_ob).wait()


_mesh = plsc.VectorSubcoreMesh(core_axis_name="c", subcore_axis_name="s")

_ce_kernel = functools.partial(
    pl.kernel,
    out_type=jax.ShapeDtypeStruct((_B, _NUM_FIELDS, _D), jnp.float32),
    mesh=_mesh,
    scratch_types=[
        pltpu.VMEM((_ROWS_PER_W,), jnp.int32),
        pltpu.VMEM((_ROWS_PER_W,), jnp.int32),
        pltpu.VMEM((_ROWS_PER_W,), jnp.int32),
        pltpu.VMEM((_CHUNK, 2 * _D), jnp.float32),
        pltpu.VMEM((_CHUNK, 2 * _D), jnp.float32),
        pltpu.VMEM((_NB, _NUM_FIELDS, _D), jnp.float32),
        pltpu.VMEM((_CHUNK, 2 * _D), jnp.float32),
        pltpu.VMEM((_CHUNK, 2 * _D), jnp.float32),
        pltpu.VMEM((_NB, _NUM_FIELDS, _D), jnp.float32),
        pltpu.SemaphoreType.DMA,
        pltpu.SemaphoreType.DMA,
        pltpu.SemaphoreType.DMA,
        pltpu.SemaphoreType.DMA,
    ],
    compiler_params=pltpu.CompilerParams(use_tc_tiling_on_sc=True),
)(_sc_body)


def kernel(x, Q_v, R_v, Q_s, R_s):
    qd, rd = _prune_tables(Q_v, Q_s, R_v, R_s)
    x_flat = x.reshape(_N)
    return _ce_kernel(x_flat, qd, rd)
